# baseline (speedup 1.0000x reference)
import functools

import jax
import jax.numpy as jnp
from jax import lax
from jax.experimental import pallas as pl
from jax.experimental.pallas import tpu as pltpu
from jax.experimental.pallas import tpu_sc as plsc

_D = 128
_NC = 2
_NS = 16
_NW = _NC * _NS


@functools.partial(jax.jit, static_argnums=(2, 3, 4))
def _gather_rows(table, flat_idx, n_rows, chunk, nbuf):
    n_per_w = n_rows // _NW
    n_chunks = n_per_w // chunk
    la = nbuf - 1
    mesh = plsc.VectorSubcoreMesh(core_axis_name="c", subcore_axis_name="s")

    @functools.partial(
        pl.kernel,
        out_type=jax.ShapeDtypeStruct((n_rows, _D), jnp.float32),
        mesh=mesh,
        scratch_types=[
            pltpu.VMEM((n_per_w,), jnp.int32),
            pltpu.VMEM((nbuf, chunk, _D), jnp.float32),
            pltpu.SemaphoreType.DMA((nbuf,)),
            pltpu.SemaphoreType.DMA((nbuf,)),
        ],
    )
    def gather_kernel(table_hbm, idx_hbm, out_hbm, idx_v, rows_v, sem_g, sem_s):
        wid = lax.axis_index("s") * _NC + lax.axis_index("c")
        base_w = pl.multiple_of(wid * n_per_w, 8)

        pltpu.sync_copy(idx_hbm.at[pl.ds(base_w, n_per_w)], idx_v)

        def start_gather(i, b):
            off = pl.multiple_of(i * chunk, 8)
            pltpu.async_copy(
                table_hbm.at[idx_v.at[pl.ds(off, chunk)]], rows_v.at[b], sem_g.at[b]
            )

        def wait_gather(b):
            pltpu.make_async_copy(
                table_hbm.at[idx_v.at[pl.ds(0, chunk)]], rows_v.at[b], sem_g.at[b]
            ).wait()

        def start_store(i, b):
            off = pl.multiple_of(base_w + i * chunk, 8)
            pltpu.async_copy(rows_v.at[b], out_hbm.at[pl.ds(off, chunk)], sem_s.at[b])

        def wait_store(b):
            pltpu.make_async_copy(
                rows_v.at[b], out_hbm.at[pl.ds(base_w, chunk)], sem_s.at[b]
            ).wait()

        for j in range(la):
            start_gather(j, j)

        wait_gather(0)
        start_store(0, 0)
        start_gather(la, nbuf - 1)

        def body(i, carry):
            b = lax.rem(i, nbuf)
            wait_gather(b)
            start_store(i, b)
            nb = lax.rem(i + la, nbuf)
            wait_store(nb)
            start_gather(i + la, nb)
            return carry

        lax.fori_loop(1, n_chunks - la, body, 0)

        def tail(i, carry):
            b = lax.rem(i, nbuf)
            wait_gather(b)
            start_store(i, b)
            return carry

        lax.fori_loop(n_chunks - la, n_chunks, tail, 0)

        def drain(i, carry):
            wait_store(lax.rem(i, nbuf))
            return carry

        lax.fori_loop(n_chunks - nbuf, n_chunks, drain, 0)

    return gather_kernel(table, flat_idx)


def kernel(inputs, table):
    b, s = inputs.shape
    n_rows = b * s
    flat_idx = inputs.reshape(n_rows).astype(jnp.int32)
    out = _gather_rows(table, flat_idx, n_rows, 320, 3)
    return out.reshape(b, s, _D)


# speedup vs baseline: 1.0170x; 1.0170x over previous
import functools

import jax
import jax.numpy as jnp
from jax import lax
from jax.experimental import pallas as pl
from jax.experimental.pallas import tpu as pltpu
from jax.experimental.pallas import tpu_sc as plsc

_D = 128
_NC = 2
_NS = 16
_NW = _NC * _NS


@functools.partial(jax.jit, static_argnums=(2, 3, 4))
def _gather_rows(table, flat_idx, n_rows, chunk, nbuf):
    n_per_w = n_rows // _NW
    n_chunks = n_per_w // chunk
    la = nbuf - 1
    mesh = plsc.VectorSubcoreMesh(core_axis_name="c", subcore_axis_name="s")

    @functools.partial(
        pl.kernel,
        out_type=jax.ShapeDtypeStruct((n_rows, _D), jnp.float32),
        mesh=mesh,
        scratch_types=[
            pltpu.VMEM((n_per_w,), jnp.int32),
            pltpu.VMEM((nbuf, chunk, _D), jnp.float32),
            pltpu.SemaphoreType.DMA((nbuf,)),
            pltpu.SemaphoreType.DMA((nbuf,)),
        ],
    )
    def gather_kernel(table_hbm, idx_hbm, out_hbm, idx_v, rows_v, sem_g, sem_s):
        wid = lax.axis_index("s") * _NC + lax.axis_index("c")
        base_w = pl.multiple_of(wid * n_per_w, 8)

        pltpu.sync_copy(idx_hbm.at[pl.ds(base_w, n_per_w)], idx_v)

        def start_gather(i, b):
            off = pl.multiple_of(i * chunk, 8)
            pltpu.async_copy(
                table_hbm.at[idx_v.at[pl.ds(off, chunk)]], rows_v.at[b], sem_g.at[b]
            )

        def wait_gather(b):
            pltpu.make_async_copy(
                table_hbm.at[idx_v.at[pl.ds(0, chunk)]], rows_v.at[b], sem_g.at[b]
            ).wait()

        def start_store(i, b):
            off = pl.multiple_of(base_w + i * chunk, 8)
            pltpu.async_copy(rows_v.at[b], out_hbm.at[pl.ds(off, chunk)], sem_s.at[b])

        def wait_store(b):
            pltpu.make_async_copy(
                rows_v.at[b], out_hbm.at[pl.ds(base_w, chunk)], sem_s.at[b]
            ).wait()

        for j in range(la):
            start_gather(j, j)

        wait_gather(0)
        start_store(0, 0)
        start_gather(la, nbuf - 1)

        def body(i, carry):
            b = lax.rem(i, nbuf)
            wait_gather(b)
            start_store(i, b)
            nb = lax.rem(i + la, nbuf)
            wait_store(nb)
            start_gather(i + la, nb)
            return carry

        lax.fori_loop(1, n_chunks - la, body, 0)

        def tail(i, carry):
            b = lax.rem(i, nbuf)
            wait_gather(b)
            start_store(i, b)
            return carry

        lax.fori_loop(n_chunks - la, n_chunks, tail, 0)

        def drain(i, carry):
            wait_store(lax.rem(i, nbuf))
            return carry

        lax.fori_loop(n_chunks - nbuf, n_chunks, drain, 0)

    return gather_kernel(table, flat_idx)


def kernel(inputs, table):
    b, s = inputs.shape
    n_rows = b * s
    flat_idx = inputs.reshape(n_rows).astype(jnp.int32)
    out = _gather_rows(table, flat_idx, n_rows, 160, 4)
    return out.reshape(b, s, _D)


# speedup vs baseline: 1.0189x; 1.0018x over previous
import functools

import jax
import jax.numpy as jnp
from jax import lax
from jax.experimental import pallas as pl
from jax.experimental.pallas import tpu as pltpu
from jax.experimental.pallas import tpu_sc as plsc

_D = 128
_NC = 2
_NS = 16
_NW = _NC * _NS


@functools.partial(jax.jit, static_argnums=(2, 3, 4))
def _gather_rows(table, flat_idx, n_rows, chunk, nbuf):
    n_per_w = n_rows // _NW
    n_chunks = n_per_w // chunk
    la = nbuf - 1
    mesh = plsc.VectorSubcoreMesh(core_axis_name="c", subcore_axis_name="s")

    @functools.partial(
        pl.kernel,
        out_type=jax.ShapeDtypeStruct((n_rows, _D), jnp.float32),
        mesh=mesh,
        scratch_types=[
            pltpu.VMEM((n_per_w,), jnp.int32),
            pltpu.VMEM((nbuf, chunk, _D), jnp.float32),
            pltpu.SemaphoreType.DMA((nbuf,)),
            pltpu.SemaphoreType.DMA((nbuf,)),
        ],
    )
    def gather_kernel(table_hbm, idx_hbm, out_hbm, idx_v, rows_v, sem_g, sem_s):
        wid = lax.axis_index("s") * _NC + lax.axis_index("c")
        base_w = pl.multiple_of(wid * n_per_w, 8)

        pltpu.sync_copy(idx_hbm.at[pl.ds(base_w, n_per_w)], idx_v)

        def start_gather(i, b):
            off = pl.multiple_of(i * chunk, 8)
            pltpu.async_copy(
                table_hbm.at[idx_v.at[pl.ds(off, chunk)]], rows_v.at[b], sem_g.at[b]
            )

        def wait_gather(b):
            pltpu.make_async_copy(
                table_hbm.at[idx_v.at[pl.ds(0, chunk)]], rows_v.at[b], sem_g.at[b]
            ).wait()

        def start_store(i, b):
            off = pl.multiple_of(base_w + i * chunk, 8)
            pltpu.async_copy(rows_v.at[b], out_hbm.at[pl.ds(off, chunk)], sem_s.at[b])

        def wait_store(b):
            pltpu.make_async_copy(
                rows_v.at[b], out_hbm.at[pl.ds(base_w, chunk)], sem_s.at[b]
            ).wait()

        for j in range(la):
            start_gather(j, j)

        wait_gather(0)
        start_store(0, 0)
        start_gather(la, nbuf - 1)

        def body(i, carry):
            b = lax.rem(i, nbuf)
            wait_gather(b)
            start_store(i, b)
            nb = lax.rem(i + la, nbuf)
            wait_store(nb)
            start_gather(i + la, nb)
            return carry

        lax.fori_loop(1, n_chunks - la, body, 0)

        def tail(i, carry):
            b = lax.rem(i, nbuf)
            wait_gather(b)
            start_store(i, b)
            return carry

        lax.fori_loop(n_chunks - la, n_chunks, tail, 0)

        def drain(i, carry):
            wait_store(lax.rem(i, nbuf))
            return carry

        lax.fori_loop(n_chunks - nbuf, n_chunks, drain, 0)

    return gather_kernel(table, flat_idx)


def kernel(inputs, table):
    b, s = inputs.shape
    n_rows = b * s
    flat_idx = inputs.reshape(n_rows).astype(jnp.int32)
    out = _gather_rows(table, flat_idx, n_rows, 128, 5)
    return out.reshape(b, s, _D)


# speedup vs baseline: 1.0256x; 1.0066x over previous
import functools

import jax
import jax.numpy as jnp
from jax import lax
from jax.experimental import pallas as pl
from jax.experimental.pallas import tpu as pltpu
from jax.experimental.pallas import tpu_sc as plsc

_D = 128
_NC = 2
_NS = 16
_NW = _NC * _NS


@functools.partial(jax.jit, static_argnums=(2, 3, 4))
def _gather_rows(table, flat_idx, n_rows, chunk, nbuf):
    n_per_w = n_rows // _NW
    n_chunks = n_per_w // chunk
    la = nbuf - 1
    mesh = plsc.VectorSubcoreMesh(core_axis_name="c", subcore_axis_name="s")

    @functools.partial(
        pl.kernel,
        out_type=jax.ShapeDtypeStruct((n_rows, _D), jnp.float32),
        mesh=mesh,
        scratch_types=[
            pltpu.VMEM((n_per_w,), jnp.int32),
            pltpu.VMEM((nbuf, chunk, _D), jnp.float32),
            pltpu.SemaphoreType.DMA((nbuf,)),
            pltpu.SemaphoreType.DMA((nbuf,)),
        ],
    )
    def gather_kernel(table_hbm, idx_hbm, out_hbm, idx_v, rows_v, sem_g, sem_s):
        wid = lax.axis_index("s") * _NC + lax.axis_index("c")
        base_w = pl.multiple_of(wid * n_per_w, 8)

        pltpu.sync_copy(idx_hbm.at[pl.ds(base_w, n_per_w)], idx_v)

        def start_gather(i, b):
            off = pl.multiple_of(i * chunk, 8)
            pltpu.async_copy(
                table_hbm.at[idx_v.at[pl.ds(off, chunk)]], rows_v.at[b], sem_g.at[b]
            )

        def wait_gather(b):
            pltpu.make_async_copy(
                table_hbm.at[idx_v.at[pl.ds(0, chunk)]], rows_v.at[b], sem_g.at[b]
            ).wait()

        def start_store(i, b):
            off = pl.multiple_of(base_w + i * chunk, 8)
            pltpu.async_copy(rows_v.at[b], out_hbm.at[pl.ds(off, chunk)], sem_s.at[b])

        def wait_store(b):
            pltpu.make_async_copy(
                rows_v.at[b], out_hbm.at[pl.ds(base_w, chunk)], sem_s.at[b]
            ).wait()

        for j in range(la):
            start_gather(j, j)

        wait_gather(0)
        start_store(0, 0)
        start_gather(la, nbuf - 1)

        def body(i, carry):
            b = lax.rem(i, nbuf)
            wait_gather(b)
            start_store(i, b)
            nb = lax.rem(i + la, nbuf)
            wait_store(nb)
            start_gather(i + la, nb)
            return carry

        lax.fori_loop(1, n_chunks - la, body, 0)

        def tail(i, carry):
            b = lax.rem(i, nbuf)
            wait_gather(b)
            start_store(i, b)
            return carry

        lax.fori_loop(n_chunks - la, n_chunks, tail, 0)

        def drain(i, carry):
            wait_store(lax.rem(i, nbuf))
            return carry

        lax.fori_loop(n_chunks - nbuf, n_chunks, drain, 0)

    return gather_kernel(table, flat_idx)


def kernel(inputs, table):
    b, s = inputs.shape
    n_rows = b * s
    flat_idx = inputs.reshape(n_rows).astype(jnp.int32)
    out = _gather_rows(table, flat_idx, n_rows, 80, 8)
    return out.reshape(b, s, _D)
